# baseline (speedup 1.0000x reference)
import jax
import jax.numpy as jnp
from jax.experimental import pallas as pl
from jax.experimental.pallas import tpu as pltpu

N = 4096
D = 256
BM = 512
BK = 2048
NI = N // BM
NK = N // BK


def _gcn_body(x_ref, adj_ref, w_ref, out_ref, support_ref):
    i = pl.program_id(0)
    k = pl.program_id(1)

    @pl.when(i == 0)
    def _compute_support():
        support_ref[pl.ds(k * BK, BK), :] = jnp.dot(
            x_ref[pl.ds(k * BK, BK), :], w_ref[...],
            preferred_element_type=jnp.float32,
        )

    partial = jnp.dot(
        adj_ref[...],
        support_ref[pl.ds(k * BK, BK), :],
        preferred_element_type=jnp.float32,
    )

    @pl.when(k == 0)
    def _init():
        out_ref[...] = partial

    @pl.when(k > 0)
    def _accum():
        out_ref[...] += partial


@jax.jit
def kernel(input, adj, W):
    grid = (NI, NK)
    return pl.pallas_call(
        _gcn_body,
        grid=grid,
        in_specs=[
            pl.BlockSpec((N, D), lambda i, k: (0, 0)),
            pl.BlockSpec((BM, BK), lambda i, k: (i, k)),
            pl.BlockSpec((D, D), lambda i, k: (0, 0)),
        ],
        out_specs=pl.BlockSpec((BM, D), lambda i, k: (i, 0)),
        out_shape=jax.ShapeDtypeStruct((N, D), jnp.float32),
        scratch_shapes=[pltpu.VMEM((N, D), jnp.float32)],
        compiler_params=pltpu.CompilerParams(
            dimension_semantics=("arbitrary", "arbitrary"),
        ),
    )(input, adj, W)


# speedup vs baseline: 1.1649x; 1.1649x over previous
import jax
import jax.numpy as jnp
from jax.experimental import pallas as pl
from jax.experimental.pallas import tpu as pltpu

N = 4096
D = 256
BM = 512
BK = 4096
NI = N // BM
NK = N // BK


def _gcn_body(adj_ref, x_ref, w_ref, out_ref, support_ref):
    i = pl.program_id(0)
    k = pl.program_id(1)

    @pl.when(i == 0)
    def _compute_support():
        support_ref[pl.ds(k * BK, BK), :] = jnp.dot(
            x_ref[...], w_ref[...], preferred_element_type=jnp.float32
        )

    partial = jnp.dot(
        adj_ref[...],
        support_ref[pl.ds(k * BK, BK), :],
        preferred_element_type=jnp.float32,
    )

    @pl.when(k == 0)
    def _init():
        out_ref[...] = partial

    @pl.when(k > 0)
    def _accum():
        out_ref[...] += partial


@jax.jit
def kernel(input, adj, W):
    grid = (NI, NK)
    return pl.pallas_call(
        _gcn_body,
        grid=grid,
        in_specs=[
            pl.BlockSpec((BM, BK), lambda i, k: (i, k)),
            pl.BlockSpec((BK, D), lambda i, k: (jnp.where(i == 0, k, NK - 1), 0)),
            pl.BlockSpec((D, D), lambda i, k: (0, 0)),
        ],
        out_specs=pl.BlockSpec((BM, D), lambda i, k: (i, 0)),
        out_shape=jax.ShapeDtypeStruct((N, D), jnp.float32),
        scratch_shapes=[pltpu.VMEM((N, D), jnp.float32)],
        compiler_params=pltpu.CompilerParams(
            dimension_semantics=("arbitrary", "arbitrary"),
        ),
    )(adj, input, W)
